# baseline (speedup 1.0000x reference)
import functools

import jax
import jax.numpy as jnp
from jax import lax
from jax.experimental import pallas as pl
from jax.experimental.pallas import tpu as pltpu
from jax.experimental.pallas import tpu_sc as plsc

V = 1000000
D = 64
B = 4096 * 200

_info = plsc.get_sparse_core_info()
NC = _info.num_cores
NS = _info.num_subcores
NW = NC * NS
IW = 128
B_PER_W = B // NW
ROWS_PER_CHUNK = 512
GATHERS_PER_CHUNK = ROWS_PER_CHUNK // IW
CHUNKS = B_PER_W // ROWS_PER_CHUNK
IDX_ROWS_PER_W = B_PER_W // IW


def _sc_gather(table_hbm, idx_hbm, out_hbm, idx_v, rows_v, gsem):
    wid = lax.axis_index("s") * NC + lax.axis_index("c")
    idx_row_base = wid * IDX_ROWS_PER_W
    out_base = wid * B_PER_W

    pltpu.sync_copy(idx_hbm.at[pl.ds(idx_row_base, IDX_ROWS_PER_W)], idx_v)

    @pl.loop(0, CHUNKS)
    def _chunk(g):
        descs = []
        for j in range(GATHERS_PER_CHUNK):
            descs.append(
                pltpu.async_copy(
                    table_hbm.at[idx_v.at[g * GATHERS_PER_CHUNK + j]],
                    rows_v.at[pl.ds(j * IW, IW)],
                    gsem,
                )
            )
        for d in descs:
            d.wait()
        pltpu.sync_copy(
            rows_v, out_hbm.at[pl.ds(out_base + g * ROWS_PER_CHUNK, ROWS_PER_CHUNK)]
        )


@jax.jit
def _lookup(indices_flat2d, table):
    mesh = plsc.VectorSubcoreMesh(core_axis_name="c", subcore_axis_name="s")
    run = pl.kernel(
        _sc_gather,
        out_type=jax.ShapeDtypeStruct((B, D), jnp.float32),
        mesh=mesh,
        scratch_types=[
            pltpu.VMEM((IDX_ROWS_PER_W, IW), jnp.int32),
            pltpu.VMEM((ROWS_PER_CHUNK, D), jnp.float32),
            pltpu.SemaphoreType.DMA,
        ],
        compiler_params=pltpu.CompilerParams(use_tc_tiling_on_sc=False),
    )
    return run(table, indices_flat2d)


def kernel(indices, table):
    idx = indices.astype(jnp.int32).reshape(B // IW, IW)
    out = _lookup(idx, table)
    return out.reshape(indices.shape + (D,))


# speedup vs baseline: 1.0268x; 1.0268x over previous
import jax
import jax.numpy as jnp
from jax import lax
from jax.experimental import pallas as pl
from jax.experimental.pallas import tpu as pltpu
from jax.experimental.pallas import tpu_sc as plsc

V = 1000000
D = 64
B = 4096 * 200

_info = plsc.get_sparse_core_info()
NC = _info.num_cores
NS = _info.num_subcores
NW = NC * NS
IW = 128
B_PER_W = B // NW
ROWS_PER_CHUNK = 512
GPC = ROWS_PER_CHUNK // IW
CHUNKS = B_PER_W // ROWS_PER_CHUNK
IDX_ROWS_PER_W = B_PER_W // IW


def _sc_gather(table_hbm, idx_hbm, out_hbm, idx_v, rows_v, gsem0, gsem1,
               ssem0, ssem1):
    wid = lax.axis_index("s") * NC + lax.axis_index("c")
    idx_row_base = wid * IDX_ROWS_PER_W
    out_base = wid * B_PER_W
    gsems = (gsem0, gsem1)
    ssems = (ssem0, ssem1)

    pltpu.sync_copy(idx_hbm.at[pl.ds(idx_row_base, IDX_ROWS_PER_W)], idx_v)

    def fire_gathers(g, b):
        for j in range(GPC):
            pltpu.async_copy(
                table_hbm.at[idx_v.at[g * GPC + j]],
                rows_v.at[b].at[pl.ds(j * IW, IW)],
                gsems[b],
            )

    def drain_gathers(b):
        pltpu.make_async_copy(
            table_hbm.at[pl.ds(0, ROWS_PER_CHUNK)], rows_v.at[b], gsems[b]
        ).wait()

    def store_async(g, b):
        pltpu.async_copy(
            rows_v.at[b],
            out_hbm.at[pl.ds(out_base + g * ROWS_PER_CHUNK, ROWS_PER_CHUNK)],
            ssems[b],
        )

    def drain_store(b):
        pltpu.make_async_copy(
            rows_v.at[b], out_hbm.at[pl.ds(out_base, ROWS_PER_CHUNK)], ssems[b]
        ).wait()

    fire_gathers(0, 0)

    @pl.loop(0, CHUNKS, step=2)
    def _pair(g):
        drain_gathers(0)
        store_async(g, 0)

        @pl.when(g > 0)
        def _():
            drain_store(1)

        fire_gathers(g + 1, 1)

        drain_gathers(1)
        store_async(g + 1, 1)

        @pl.when(g + 2 < CHUNKS)
        def _():
            drain_store(0)
            fire_gathers(g + 2, 0)

    drain_store(0)
    drain_store(1)


@jax.jit
def _lookup(indices_flat2d, table):
    mesh = plsc.VectorSubcoreMesh(core_axis_name="c", subcore_axis_name="s")
    run = pl.kernel(
        _sc_gather,
        out_type=jax.ShapeDtypeStruct((B, D), jnp.float32),
        mesh=mesh,
        scratch_types=[
            pltpu.VMEM((IDX_ROWS_PER_W, IW), jnp.int32),
            pltpu.VMEM((2, ROWS_PER_CHUNK, D), jnp.float32),
            pltpu.SemaphoreType.DMA,
            pltpu.SemaphoreType.DMA,
            pltpu.SemaphoreType.DMA,
            pltpu.SemaphoreType.DMA,
        ],
        compiler_params=pltpu.CompilerParams(use_tc_tiling_on_sc=False),
    )
    return run(table, indices_flat2d)


def kernel(indices, table):
    idx = indices.astype(jnp.int32).reshape(B // IW, IW)
    out = _lookup(idx, table)
    return out.reshape(indices.shape + (D,))
